# baseline (speedup 1.0000x reference)
import jax
import jax.numpy as jnp
from jax.experimental import pallas as pl
from jax.experimental.pallas import tpu as pltpu

T = 2048
H = 1024
E = 8
I = 512
ISH = 1024
ROUTED_SCALING = 2.5

NQ = 4
Q = T // NQ


def _dot_t(a, b):
    return jax.lax.dot_general(
        a, b, (((1,), (1,)), ((), ())), preferred_element_type=jnp.float32)


def _argmax8_lowest(vals):
    best = vals[:, 0:1]
    bidx = jnp.zeros_like(best, dtype=jnp.int32)
    for c in range(1, E):
        v = vals[:, c : c + 1]
        take = v > best
        bidx = jnp.where(take, jnp.int32(c), bidx)
        best = jnp.where(take, v, best)
    return bidx


def _routing_weights(xb, gate_w, e_bias):
    logits = _dot_t(xb, gate_w.astype(jnp.bfloat16))
    scores = jax.nn.sigmoid(logits)
    s4c = scores + e_bias

    def top2sum(g):
        cols = [g[:, c : c + 1] for c in range(4)]
        ps = [cols[i] + cols[j] for i in range(4) for j in range(i + 1, 4)]
        out = ps[0]
        for p in ps[1:]:
            out = jnp.maximum(out, p)
        return out

    one = jnp.float32(1.0)
    zero = jnp.float32(0.0)
    use_g0 = jnp.where(top2sum(s4c[:, 0:4]) >= top2sum(s4c[:, 4:8]), one, zero)
    col = jax.lax.broadcasted_iota(jnp.int32, (Q, E), 1)
    in_g0 = jnp.where(col < 4, one, zero)
    group_mask = in_g0 * use_g0 + (one - in_g0) * (one - use_g0)
    neg_inf = jnp.float32(-jnp.inf)
    masked = jnp.where(group_mask > 0.5, s4c, neg_inf)

    idx1 = _argmax8_lowest(masked)
    idx2 = _argmax8_lowest(jnp.where(col == idx1, neg_inf, masked))
    oh1 = jnp.where(col == idx1, one, zero)
    oh2 = jnp.where(col == idx2, one, zero)
    w1 = jnp.sum(oh1 * scores, axis=1, keepdims=True)
    w2 = jnp.sum(oh2 * scores, axis=1, keepdims=True)
    return (oh1 * w1 + oh2 * w2) / (w1 + w2 + jnp.float32(1e-20))


def _moe_kernel(x_ref, gate_w_ref, e_bias_ref, wg_ref, wu_ref, wd_ref,
                sg_ref, su_ref, sd_ref, out_ref, xb_ref, we_ref):
    s = pl.program_id(0)
    j = pl.program_id(1)
    row = j * Q

    @pl.when(s == 0)
    def _prologue():
        xb = x_ref[pl.ds(row, Q), :].astype(jnp.bfloat16)
        xb_ref[pl.ds(row, Q), :] = xb
        we_ref[pl.ds(row, Q), :] = _routing_weights(
            xb, gate_w_ref[...], e_bias_ref[...])
        sg = _dot_t(xb, sg_ref[...])
        su = _dot_t(xb, su_ref[...])
        sh = (jax.nn.silu(sg) * su).astype(jnp.bfloat16)
        out_ref[pl.ds(row, Q), :] = _dot_t(sh, sd_ref[...])

    @pl.when(s > 0)
    def _expert():
        e = s - 1
        xb = xb_ref[pl.ds(row, Q), :]
        wg = wg_ref[0].astype(jnp.bfloat16)
        wu = wu_ref[0].astype(jnp.bfloat16)
        wd = wd_ref[0].astype(jnp.bfloat16)
        g = _dot_t(xb, wg)
        u = _dot_t(xb, wu)
        h = (jax.nn.silu(g) * u).astype(jnp.bfloat16)
        d = _dot_t(h, wd)
        col = jax.lax.broadcasted_iota(jnp.int32, (Q, E), 1)
        wsel = jnp.sum(
            jnp.where(col == e, we_ref[pl.ds(row, Q), :], jnp.float32(0.0)),
            axis=1, keepdims=True)
        out_ref[pl.ds(row, Q), :] += jnp.float32(ROUTED_SCALING) * wsel * d


def kernel(hidden_states, gate_w, e_bias, w_gate, w_up, w_down,
           sw_gate, sw_up, sw_down):
    e_bias2 = e_bias.reshape(1, E)
    const2 = lambda s, j: (0, 0)
    expert_ix = lambda s, j: (jnp.maximum(s - 1, 0), 0, 0)
    out = pl.pallas_call(
        _moe_kernel,
        grid=(1 + E, NQ),
        in_specs=[
            pl.BlockSpec((T, H), const2),
            pl.BlockSpec((E, H), const2),
            pl.BlockSpec((1, E), const2),
            pl.BlockSpec((1, I, H), expert_ix),
            pl.BlockSpec((1, I, H), expert_ix),
            pl.BlockSpec((1, H, I), expert_ix),
            pl.BlockSpec((ISH, H), const2),
            pl.BlockSpec((ISH, H), const2),
            pl.BlockSpec((H, ISH), const2),
        ],
        out_specs=pl.BlockSpec((T, H), const2),
        out_shape=jax.ShapeDtypeStruct((T, H), jnp.float32),
        scratch_shapes=[
            pltpu.VMEM((T, H), jnp.bfloat16),
            pltpu.VMEM((T, E), jnp.float32),
        ],
        compiler_params=pltpu.CompilerParams(
            dimension_semantics=("arbitrary", "arbitrary"),
        ),
    )(hidden_states, gate_w, e_bias2, w_gate, w_up, w_down,
      sw_gate, sw_up, sw_down)
    return out


# speedup vs baseline: 1.1692x; 1.1692x over previous
import jax
import jax.numpy as jnp
from jax.experimental import pallas as pl
from jax.experimental.pallas import tpu as pltpu

T = 2048
H = 1024
E = 8
I = 512
ISH = 1024
ROUTED_SCALING = 2.5

NQ = 2
Q = T // NQ


def _dot_t(a, b):
    return jax.lax.dot_general(
        a, b, (((1,), (1,)), ((), ())), preferred_element_type=jnp.float32)


def _argmax8_lowest(vals):
    best = vals[:, 0:1]
    bidx = jnp.zeros_like(best, dtype=jnp.int32)
    for c in range(1, E):
        v = vals[:, c : c + 1]
        take = v > best
        bidx = jnp.where(take, jnp.int32(c), bidx)
        best = jnp.where(take, v, best)
    return bidx


def _routing_weights(xb, gate_w, e_bias):
    logits = _dot_t(xb, gate_w.astype(jnp.bfloat16))
    scores = jax.nn.sigmoid(logits)
    s4c = scores + e_bias

    def top2sum(g):
        cols = [g[:, c : c + 1] for c in range(4)]
        ps = [cols[i] + cols[j] for i in range(4) for j in range(i + 1, 4)]
        out = ps[0]
        for p in ps[1:]:
            out = jnp.maximum(out, p)
        return out

    one = jnp.float32(1.0)
    zero = jnp.float32(0.0)
    use_g0 = jnp.where(top2sum(s4c[:, 0:4]) >= top2sum(s4c[:, 4:8]), one, zero)
    col = jax.lax.broadcasted_iota(jnp.int32, (Q, E), 1)
    in_g0 = jnp.where(col < 4, one, zero)
    group_mask = in_g0 * use_g0 + (one - in_g0) * (one - use_g0)
    neg_inf = jnp.float32(-jnp.inf)
    masked = jnp.where(group_mask > 0.5, s4c, neg_inf)

    idx1 = _argmax8_lowest(masked)
    idx2 = _argmax8_lowest(jnp.where(col == idx1, neg_inf, masked))
    oh1 = jnp.where(col == idx1, one, zero)
    oh2 = jnp.where(col == idx2, one, zero)
    w1 = jnp.sum(oh1 * scores, axis=1, keepdims=True)
    w2 = jnp.sum(oh2 * scores, axis=1, keepdims=True)
    return (oh1 * w1 + oh2 * w2) / (w1 + w2 + jnp.float32(1e-20))


def _moe_kernel(x_ref, gate_w_ref, e_bias_ref, wg_ref, wu_ref, wd_ref,
                sg_ref, su_ref, sd_ref, out_ref, xb_ref, we_ref):
    s = pl.program_id(0)
    j = pl.program_id(1)
    row = j * Q

    @pl.when(s == 0)
    def _prologue():
        xb = x_ref[pl.ds(row, Q), :].astype(jnp.bfloat16)
        xb_ref[pl.ds(row, Q), :] = xb
        we_ref[pl.ds(row, Q), :] = _routing_weights(
            xb, gate_w_ref[...], e_bias_ref[...])
        sg = _dot_t(xb, sg_ref[...])
        su = _dot_t(xb, su_ref[...])
        sh = (jax.nn.silu(sg) * su).astype(jnp.bfloat16)
        out_ref[pl.ds(row, Q), :] = _dot_t(sh, sd_ref[...])

    @pl.when(s > 0)
    def _expert():
        e = s - 1
        xb = xb_ref[pl.ds(row, Q), :]
        wg = wg_ref[0].astype(jnp.bfloat16)
        wu = wu_ref[0].astype(jnp.bfloat16)
        wd = wd_ref[0].astype(jnp.bfloat16)
        g = _dot_t(xb, wg)
        u = _dot_t(xb, wu)
        h = (jax.nn.silu(g) * u).astype(jnp.bfloat16)
        d = _dot_t(h, wd)
        col = jax.lax.broadcasted_iota(jnp.int32, (Q, E), 1)
        wsel = jnp.sum(
            jnp.where(col == e, we_ref[pl.ds(row, Q), :], jnp.float32(0.0)),
            axis=1, keepdims=True)
        out_ref[pl.ds(row, Q), :] += jnp.float32(ROUTED_SCALING) * wsel * d


def kernel(hidden_states, gate_w, e_bias, w_gate, w_up, w_down,
           sw_gate, sw_up, sw_down):
    e_bias2 = e_bias.reshape(1, E)
    const2 = lambda s, j: (0, 0)
    expert_ix = lambda s, j: (jnp.maximum(s - 1, 0), 0, 0)
    out = pl.pallas_call(
        _moe_kernel,
        grid=(1 + E, NQ),
        in_specs=[
            pl.BlockSpec((T, H), const2),
            pl.BlockSpec((E, H), const2),
            pl.BlockSpec((1, E), const2),
            pl.BlockSpec((1, I, H), expert_ix),
            pl.BlockSpec((1, I, H), expert_ix),
            pl.BlockSpec((1, H, I), expert_ix),
            pl.BlockSpec((ISH, H), const2),
            pl.BlockSpec((ISH, H), const2),
            pl.BlockSpec((H, ISH), const2),
        ],
        out_specs=pl.BlockSpec((T, H), const2),
        out_shape=jax.ShapeDtypeStruct((T, H), jnp.float32),
        scratch_shapes=[
            pltpu.VMEM((T, H), jnp.bfloat16),
            pltpu.VMEM((T, E), jnp.float32),
        ],
        compiler_params=pltpu.CompilerParams(
            dimension_semantics=("arbitrary", "arbitrary"),
        ),
    )(hidden_states, gate_w, e_bias2, w_gate, w_up, w_down,
      sw_gate, sw_up, sw_down)
    return out
